# baseline (speedup 1.0000x reference)
import functools

import jax
import jax.numpy as jnp
from jax.experimental import pallas as pl
from jax.experimental.pallas import tpu as pltpu

_INV_SQRT_2PI = 0.3989422804014327


def _cppn_body(x_ref, y_ref, z_ref, b_ref, a_ref, w_ref, out_ref):
    w10 = w_ref[0]
    w11 = w_ref[1]
    w20 = w_ref[2]
    w21 = w_ref[3]
    w22 = w_ref[4]

    s = jnp.dot(z_ref[...], b_ref[...], preferred_element_type=jnp.float32)
    pre1 = w10 * x_ref[...] + w11 * y_ref[...] + s[:, :128]
    h1 = jnp.sin(pre1)
    pre2 = w20 * x_ref[...] + w21 * y_ref[...] + w22 * h1 + s[:, 128:]
    h2 = _INV_SQRT_2PI * jnp.exp(-0.5 * pre2 * pre2)
    h = jnp.concatenate([h1, h2], axis=1)
    p = jnp.dot(h, a_ref[...], preferred_element_type=jnp.float32)
    out_ref[...] = 1.0 / (1.0 + jnp.exp(-p))


@jax.jit
def _run(X, Y, Z, w1, w2, w_out):
    N = X.shape[0]
    M = N // 128
    BM = 256

    Xr = X.reshape(M, 128)
    Yr = Y.reshape(M, 128)
    Zr = Z.reshape(M, 2048)

    eye = jnp.eye(128, dtype=jnp.float32)
    b1 = (eye[:, None, :] * w1[2:][None, :, None]).reshape(2048, 128)
    b2 = (eye[:, None, :] * w2[3:][None, :, None]).reshape(2048, 128)
    B = jnp.concatenate([b1, b2], axis=1)

    a1 = (eye[:, :, None] * w_out[0][None, None, :]).reshape(128, 384)
    a2 = (eye[:, :, None] * w_out[1][None, None, :]).reshape(128, 384)
    A = jnp.concatenate([a1, a2], axis=0)

    wsc = jnp.stack([w1[0], w1[1], w2[0], w2[1], w2[2]])

    out = pl.pallas_call(
        _cppn_body,
        grid=(M // BM,),
        in_specs=[
            pl.BlockSpec((BM, 128), lambda i: (i, 0)),
            pl.BlockSpec((BM, 128), lambda i: (i, 0)),
            pl.BlockSpec((BM, 2048), lambda i: (i, 0)),
            pl.BlockSpec((2048, 256), lambda i: (0, 0)),
            pl.BlockSpec((256, 384), lambda i: (0, 0)),
            pl.BlockSpec(memory_space=pltpu.SMEM),
        ],
        out_specs=pl.BlockSpec((BM, 384), lambda i: (i, 0)),
        out_shape=jax.ShapeDtypeStruct((M, 384), jnp.float32),
    )(Xr, Yr, Zr, B, A, wsc)
    return out.reshape(N, 3)


def kernel(X, Y, R, Z, w1, w2, w_out):
    del R
    return _run(X, Y, Z, w1, w2, w_out)


# speedup vs baseline: 12.9519x; 12.9519x over previous
import jax
import jax.numpy as jnp
from jax.experimental import pallas as pl
from jax.experimental.pallas import tpu as pltpu

_INV_SQRT_2PI = 0.3989422804014327


def _cppn_body(w_ref, x_ref, y_ref, z_ref, wz_ref, out_ref):
    x = x_ref[...]
    y = y_ref[...]
    zb0 = z_ref[0]
    zb1 = z_ref[1]
    s1 = (w_ref[0] * x + w_ref[1] * y
          + jnp.sum(zb0 * wz_ref[0], axis=1)
          + jnp.sum(zb1 * wz_ref[1], axis=1))
    s2 = (w_ref[2] * x + w_ref[3] * y
          + jnp.sum(zb0 * wz_ref[2], axis=1)
          + jnp.sum(zb1 * wz_ref[3], axis=1))
    h1 = jnp.sin(s1)
    pre2 = s2 + w_ref[4] * h1
    h2 = _INV_SQRT_2PI * jnp.exp(-0.5 * pre2 * pre2)
    for j in range(3):
        p = w_ref[5 + j] * h1 + w_ref[8 + j] * h2
        out_ref[j] = 1.0 / (1.0 + jnp.exp(-p))


@jax.jit
def _run(X, Y, Z, w1, w2, w_out):
    N = X.shape[0]
    M = N // 128
    BM = 256

    Xr = X.reshape(M, 128)
    Yr = Y.reshape(M, 128)
    Zr = Z.reshape(M, 128, 2, 8).transpose(2, 0, 3, 1)

    wsc = jnp.concatenate([
        jnp.stack([w1[0], w1[1], w2[0], w2[1], w2[2]]),
        w_out[0], w_out[1],
    ])
    wz = jnp.broadcast_to(
        jnp.concatenate([w1[2:], w2[3:]]).reshape(4, 8, 1), (4, 8, 128))

    out3 = pl.pallas_call(
        _cppn_body,
        grid=(M // BM,),
        in_specs=[
            pl.BlockSpec(memory_space=pltpu.SMEM),
            pl.BlockSpec((BM, 128), lambda i: (i, 0)),
            pl.BlockSpec((BM, 128), lambda i: (i, 0)),
            pl.BlockSpec((2, BM, 8, 128), lambda i: (0, i, 0, 0)),
            pl.BlockSpec((4, 8, 128), lambda i: (0, 0, 0)),
        ],
        out_specs=pl.BlockSpec((3, BM, 128), lambda i: (0, i, 0)),
        out_shape=jax.ShapeDtypeStruct((3, M, 128), jnp.float32),
    )(wsc, Xr, Yr, Zr, wz)
    return out3.reshape(3, N).T


def kernel(X, Y, R, Z, w1, w2, w_out):
    del R
    return _run(X, Y, Z, w1, w2, w_out)


# speedup vs baseline: 13.9478x; 1.0769x over previous
import jax
import jax.numpy as jnp
from jax.experimental import pallas as pl
from jax.experimental.pallas import tpu as pltpu

_INV_SQRT_2PI = 0.3989422804014327


def _cppn_body(w_ref, x_ref, y_ref, z_ref, wz_ref, out_ref):
    x = x_ref[...]
    y = y_ref[...]
    zb0 = z_ref[0]
    zb1 = z_ref[1]
    s1 = (w_ref[0] * x + w_ref[1] * y
          + jnp.sum(zb0 * wz_ref[0] + zb1 * wz_ref[1], axis=1))
    s2 = (w_ref[2] * x + w_ref[3] * y
          + jnp.sum(zb0 * wz_ref[2] + zb1 * wz_ref[3], axis=1))
    h1 = jnp.sin(s1)
    pre2 = s2 + w_ref[4] * h1
    h2 = _INV_SQRT_2PI * jnp.exp(-0.5 * pre2 * pre2)
    for j in range(3):
        p = w_ref[5 + j] * h1 + w_ref[8 + j] * h2
        out_ref[j] = 1.0 / (1.0 + jnp.exp(-p))


@jax.jit
def _run(X, Y, Z, w1, w2, w_out):
    N = X.shape[0]
    M = N // 128
    BM = 256

    Xr = X.reshape(M, 128)
    Yr = Y.reshape(M, 128)
    Zr = Z.reshape(M, 128, 2, 8).transpose(2, 0, 3, 1)

    wsc = jnp.concatenate([
        jnp.stack([w1[0], w1[1], w2[0], w2[1], w2[2]]),
        w_out[0], w_out[1],
    ])
    wz = jnp.broadcast_to(
        jnp.concatenate([w1[2:], w2[3:]]).reshape(4, 8, 1), (4, 8, 128))

    out3 = pl.pallas_call(
        _cppn_body,
        grid=(M // BM,),
        in_specs=[
            pl.BlockSpec(memory_space=pltpu.SMEM),
            pl.BlockSpec((BM, 128), lambda i: (i, 0)),
            pl.BlockSpec((BM, 128), lambda i: (i, 0)),
            pl.BlockSpec((2, BM, 8, 128), lambda i: (0, i, 0, 0)),
            pl.BlockSpec((4, 8, 128), lambda i: (0, 0, 0)),
        ],
        out_specs=pl.BlockSpec((3, BM, 128), lambda i: (0, i, 0)),
        out_shape=jax.ShapeDtypeStruct((3, M, 128), jnp.float32),
    )(wsc, Xr, Yr, Zr, wz)
    return out3.reshape(3, N).T


def kernel(X, Y, R, Z, w1, w2, w_out):
    del R
    return _run(X, Y, Z, w1, w2, w_out)


# speedup vs baseline: 20.9406x; 1.5014x over previous
import jax
import jax.numpy as jnp
from jax.experimental import pallas as pl
from jax.experimental.pallas import tpu as pltpu

_INV_SQRT_2PI = 0.3989422804014327


def _row(w_ref, i):
    return jnp.full((1, 128), w_ref[i], dtype=jnp.float32)


def _cppn_body(w1_ref, w2_ref, wo_ref, x_ref, y_ref, z_ref, out_ref, wz_ref):
    @pl.when(pl.program_id(0) == 0)
    def _init():
        wz_ref[0] = jnp.concatenate([_row(w1_ref, 2 + s) for s in range(8)], 0)
        wz_ref[1] = jnp.concatenate([_row(w1_ref, 10 + s) for s in range(8)], 0)
        wz_ref[2] = jnp.concatenate([_row(w2_ref, 3 + s) for s in range(8)], 0)
        wz_ref[3] = jnp.concatenate([_row(w2_ref, 11 + s) for s in range(8)], 0)

    x = x_ref[...]
    y = y_ref[...]
    zb0 = z_ref[0]
    zb1 = z_ref[1]
    s1 = (w1_ref[0] * x + w1_ref[1] * y
          + jnp.sum(zb0 * wz_ref[0] + zb1 * wz_ref[1], axis=1))
    s2 = (w2_ref[0] * x + w2_ref[1] * y
          + jnp.sum(zb0 * wz_ref[2] + zb1 * wz_ref[3], axis=1))
    h1 = jnp.sin(s1)
    pre2 = s2 + w2_ref[2] * h1
    h2 = _INV_SQRT_2PI * jnp.exp(-0.5 * pre2 * pre2)
    o = []
    for j in range(3):
        p = wo_ref[0, j] * h1 + wo_ref[1, j] * h2
        o.append(1.0 / (1.0 + jnp.exp(-p)))
    o.append(o[2])
    out_ref[...] = jnp.stack(o, axis=1).reshape(out_ref.shape)


@jax.jit
def _run(X, Y, Z, w1, w2, w_out):
    N = X.shape[0]
    M = N // 128
    BM = 256

    Xr = X.reshape(M, 128)
    Yr = Y.reshape(M, 128)
    Zr = Z.reshape(M, 128, 2, 8).transpose(2, 0, 3, 1)

    out4 = pl.pallas_call(
        _cppn_body,
        grid=(M // BM,),
        in_specs=[
            pl.BlockSpec(memory_space=pltpu.SMEM),
            pl.BlockSpec(memory_space=pltpu.SMEM),
            pl.BlockSpec(memory_space=pltpu.SMEM),
            pl.BlockSpec((BM, 128), lambda i: (i, 0)),
            pl.BlockSpec((BM, 128), lambda i: (i, 0)),
            pl.BlockSpec((2, BM, 8, 128), lambda i: (0, i, 0, 0)),
        ],
        out_specs=pl.BlockSpec((4 * BM, 128), lambda i: (i, 0)),
        out_shape=jax.ShapeDtypeStruct((4 * M, 128), jnp.float32),
        scratch_shapes=[pltpu.VMEM((4, 8, 128), jnp.float32)],
    )(w1, w2, w_out, Xr, Yr, Zr)
    return out4.reshape(M, 4, 128).transpose(0, 2, 1).reshape(N, 4)[:, :3]


def kernel(X, Y, R, Z, w1, w2, w_out):
    del R
    return _run(X, Y, Z, w1, w2, w_out)
